# Initial kernel scaffold; baseline (speedup 1.0000x reference)
import jax
import jax.numpy as jnp
from jax.experimental import pallas as pl


def kernel(params, default, flat_pos, indices):
    raise NotImplementedError("write your pallas kernel here")


# speedup vs baseline: 5.0084x; 5.0084x over previous
import functools

import jax
import jax.numpy as jnp
from jax import lax
from jax.experimental import pallas as pl
from jax.experimental.pallas import tpu as pltpu
from jax.experimental.pallas import tpu_sc as plsc

NC = 2
NS = 16
NW = NC * NS
CHUNK = 128

COPY_ROWS = 256


def _tc_copy_body(src, dst):
    dst[...] = src[...]


def _sc_scatter_body(pc, pos_hbm, ind_hbm, params_hbm, out_ref,
                     pos_v, ind_v, vals_v):
    c = lax.axis_index("c")
    s = lax.axis_index("s")
    wid = s * NC + c
    pltpu.sync_copy(pos_hbm.at[wid], pos_v)
    pltpu.sync_copy(ind_hbm.at[wid], ind_v)

    def body(j, carry):
        pltpu.sync_copy(params_hbm.at[ind_v.at[j]], vals_v.at[j])
        pltpu.sync_copy(vals_v.at[j], out_ref.at[pos_v.at[j]])
        return carry

    lax.fori_loop(0, pc, body, 0)


def kernel(params, default, flat_pos, indices):
    n_rows, n_cols = default.shape
    nnz = flat_pos.shape[0]
    pc = -(-nnz // (NW * CHUNK))
    padded = NW * pc * CHUNK

    pad = padded - nnz
    pos_p = jnp.concatenate(
        [flat_pos, jnp.broadcast_to(flat_pos[-1:], (pad,))]
    ).reshape(NW, pc, CHUNK)
    ind_p = jnp.concatenate(
        [indices, jnp.broadcast_to(indices[-1:], (pad,))]
    ).reshape(NW, pc, CHUNK)

    copied = pl.pallas_call(
        _tc_copy_body,
        grid=(n_rows // COPY_ROWS,),
        in_specs=[pl.BlockSpec((COPY_ROWS, n_cols), lambda i: (i, 0))],
        out_specs=pl.BlockSpec((COPY_ROWS, n_cols), lambda i: (i, 0)),
        out_shape=jax.ShapeDtypeStruct((n_rows, n_cols), default.dtype),
    )(default)

    out_ref = jax.new_ref(copied.reshape(-1))

    mesh = plsc.VectorSubcoreMesh(
        core_axis_name="c", subcore_axis_name="s",
        num_cores=NC, num_subcores=NS,
    )
    scatter = pl.kernel(
        functools.partial(_sc_scatter_body, pc),
        out_type=(),
        mesh=mesh,
        scratch_types=[
            pltpu.VMEM((pc, CHUNK), jnp.int32),
            pltpu.VMEM((pc, CHUNK), jnp.int32),
            pltpu.VMEM((pc, CHUNK), jnp.float32),
        ],
    )
    scatter(pos_p, ind_p, params, out_ref)

    return jax.freeze(out_ref).reshape(n_rows, n_cols)
